# baseline (speedup 1.0000x reference)
import functools

import jax
import jax.numpy as jnp
from jax import lax
from jax.experimental import pallas as pl
from jax.experimental.pallas import tpu as pltpu
from jax.experimental.pallas import tpu_sc as plsc

N = 10000
E = 160000
AF = 256
NF = 16
H = 2 * AF

NC = 2
NS = 16
NW = NC * NS
EW = E // NW
CHUNK = 40
NCHUNK = EW // CHUNK
Z_PAD = 10240
ROWS_PER_TILE = Z_PAD // NS


def _proj_body(a_ref, w1_ref, w2_ref, o1_ref, o2_ref):
    a = a_ref[...]
    o1_ref[...] = jnp.dot(a, w1_ref[...], preferred_element_type=jnp.float32)
    o2_ref[...] = jnp.dot(a, w2_ref[...], preferred_element_type=jnp.float32)


def _project(a, w1, w2):
    bn = 1000
    return pl.pallas_call(
        _proj_body,
        grid=(N // bn,),
        in_specs=[
            pl.BlockSpec((bn, AF), lambda i: (i, 0)),
            pl.BlockSpec((AF, H), lambda i: (0, 0)),
            pl.BlockSpec((AF, H), lambda i: (0, 0)),
        ],
        out_specs=[
            pl.BlockSpec((bn, H), lambda i: (i, 0)),
            pl.BlockSpec((bn, H), lambda i: (i, 0)),
        ],
        out_shape=[
            jax.ShapeDtypeStruct((N, H), jnp.float32),
            jax.ShapeDtypeStruct((N, H), jnp.float32),
        ],
    )(a, w1, w2)


def _gather_body(p1_hbm, p2_hbm, row_hbm, col_hbm, g1_hbm, g2_hbm,
                 idx1_v, idx2_v, buf1_v, buf2_v, sem1, sem2):
    cid = lax.axis_index("c")
    sid = lax.axis_index("s")
    wid = sid * NC + cid
    base0 = wid * EW

    @pl.loop(0, NCHUNK)
    def _(i):
        base = base0 + i * CHUNK
        pltpu.sync_copy(row_hbm.at[pl.ds(base, CHUNK)], idx1_v)
        pltpu.sync_copy(col_hbm.at[pl.ds(base, CHUNK)], idx2_v)
        d1 = pltpu.async_copy(p1_hbm.at[idx1_v], buf1_v, sem1)
        d2 = pltpu.async_copy(p2_hbm.at[idx2_v], buf2_v, sem2)
        d1.wait()
        pltpu.sync_copy(buf1_v, g1_hbm.at[pl.ds(base, CHUNK)])
        d2.wait()
        pltpu.sync_copy(buf2_v, g2_hbm.at[pl.ds(base, CHUNK)])


def _edge_gather(p1, p2, row, col):
    mesh = plsc.VectorSubcoreMesh(core_axis_name="c", subcore_axis_name="s")
    return pl.kernel(
        _gather_body,
        out_type=[
            jax.ShapeDtypeStruct((E, H), jnp.float32),
            jax.ShapeDtypeStruct((E, H), jnp.float32),
        ],
        mesh=mesh,
        scratch_types=[
            pltpu.VMEM((CHUNK,), jnp.int32),
            pltpu.VMEM((CHUNK,), jnp.int32),
            pltpu.VMEM((CHUNK, H), jnp.float32),
            pltpu.VMEM((CHUNK, H), jnp.float32),
            pltpu.SemaphoreType.DMA,
            pltpu.SemaphoreType.DMA,
        ],
    )(p1, p2, row, col)


def _fuse_body(g1_ref, g2_ref, nbr_ref, w3_ref, b_ref, x_ref, s_ref, q_ref):
    x = g1_ref[...] + g2_ref[...]
    x = x + jnp.dot(nbr_ref[...], w3_ref[...], preferred_element_type=jnp.float32)
    x = x + b_ref[...]
    x_ref[...] = x

    @pl.when(pl.program_id(0) == 0)
    def _():
        s_ref[...] = jnp.zeros_like(s_ref)
        q_ref[...] = jnp.zeros_like(q_ref)

    s_ref[...] += jnp.sum(x, axis=0, keepdims=True)
    q_ref[...] += jnp.sum(x * x, axis=0, keepdims=True)


def _fuse_stats(g1, g2, nbr, w3, b):
    be = 1000
    return pl.pallas_call(
        _fuse_body,
        grid=(E // be,),
        in_specs=[
            pl.BlockSpec((be, H), lambda i: (i, 0)),
            pl.BlockSpec((be, H), lambda i: (i, 0)),
            pl.BlockSpec((be, NF), lambda i: (i, 0)),
            pl.BlockSpec((NF, H), lambda i: (0, 0)),
            pl.BlockSpec((1, H), lambda i: (0, 0)),
        ],
        out_specs=[
            pl.BlockSpec((be, H), lambda i: (i, 0)),
            pl.BlockSpec((1, H), lambda i: (0, 0)),
            pl.BlockSpec((1, H), lambda i: (0, 0)),
        ],
        out_shape=[
            jax.ShapeDtypeStruct((E, H), jnp.float32),
            jax.ShapeDtypeStruct((1, H), jnp.float32),
            jax.ShapeDtypeStruct((1, H), jnp.float32),
        ],
    )(g1, g2, nbr, w3, b)


def _softplus(x):
    return jnp.maximum(x, 0.0) + jnp.log1p(jnp.exp(-jnp.abs(x)))


def _apply_body(x_ref, s_ref, q_ref, gm_ref, bt_ref, y_ref):
    mean = s_ref[...] / E
    var = q_ref[...] / E - mean * mean
    scale = gm_ref[...] * lax.rsqrt(var + 1e-5)
    shift = bt_ref[...] - mean * scale
    xb = x_ref[...] * scale + shift
    filt = jax.nn.sigmoid(xb[:, :AF])
    core = _softplus(xb[:, AF:])
    y_ref[...] = filt * core


def _apply_act(x, s, q, gamma1, beta1):
    be = 1000
    return pl.pallas_call(
        _apply_body,
        grid=(E // be,),
        in_specs=[
            pl.BlockSpec((be, H), lambda i: (i, 0)),
            pl.BlockSpec((1, H), lambda i: (0, 0)),
            pl.BlockSpec((1, H), lambda i: (0, 0)),
            pl.BlockSpec((1, H), lambda i: (0, 0)),
            pl.BlockSpec((1, H), lambda i: (0, 0)),
        ],
        out_specs=pl.BlockSpec((be, AF), lambda i: (i, 0)),
        out_shape=jax.ShapeDtypeStruct((E, AF), jnp.float32),
    )(x, s, q, gamma1, beta1)


def _scatter_body(y_hbm, col_hbm, zeros_hbm, z_hbm, acc_sh, idx_v, ybuf_v):
    cid = lax.axis_index("c")
    sid = lax.axis_index("s")
    wid = sid * NC + cid
    base0 = wid * EW

    pltpu.sync_copy(zeros_hbm, acc_sh.at[pl.ds(sid * ROWS_PER_TILE, ROWS_PER_TILE)])
    plsc.subcore_barrier()

    @pl.when(sid == 0)
    def _():
        @pl.loop(0, E // CHUNK)
        def _(i):
            base = i * CHUNK
            pltpu.sync_copy(col_hbm.at[pl.ds(base, CHUNK)], idx_v)
            pltpu.sync_copy(
                y_hbm.at[pl.ds(base, CHUNK), pl.ds(cid * (AF // NC), AF // NC)],
                ybuf_v)
            pltpu.sync_copy(ybuf_v, acc_sh.at[idx_v], add=True)

    plsc.subcore_barrier()
    pltpu.sync_copy(
        acc_sh.at[pl.ds(sid * ROWS_PER_TILE, ROWS_PER_TILE)],
        z_hbm.at[pl.ds(sid * ROWS_PER_TILE, ROWS_PER_TILE),
                 pl.ds(cid * (AF // NC), AF // NC)],
    )


def _segment_sum(y, col):
    mesh = plsc.VectorSubcoreMesh(core_axis_name="c", subcore_axis_name="s")
    zeros = jnp.zeros((ROWS_PER_TILE, AF // NC), jnp.float32)
    return pl.kernel(
        _scatter_body,
        out_type=jax.ShapeDtypeStruct((Z_PAD, AF), jnp.float32),
        mesh=mesh,
        scratch_types=[
            pltpu.VMEM_SHARED((Z_PAD, AF // NC), jnp.float32),
            pltpu.VMEM((CHUNK,), jnp.int32),
            pltpu.VMEM((CHUNK, AF // NC), jnp.float32),
        ],
    )(y, col, zeros)


def _zstat_body(z_ref, s_ref, q_ref):
    z = z_ref[...]

    @pl.when(pl.program_id(0) == 0)
    def _():
        s_ref[...] = jnp.zeros_like(s_ref)
        q_ref[...] = jnp.zeros_like(q_ref)

    s_ref[...] += jnp.sum(z, axis=0, keepdims=True)
    q_ref[...] += jnp.sum(z * z, axis=0, keepdims=True)


def _z_stats(z):
    bn = 1000
    return pl.pallas_call(
        _zstat_body,
        grid=(N // bn,),
        in_specs=[pl.BlockSpec((bn, AF), lambda i: (i, 0))],
        out_specs=[
            pl.BlockSpec((1, AF), lambda i: (0, 0)),
            pl.BlockSpec((1, AF), lambda i: (0, 0)),
        ],
        out_shape=[
            jax.ShapeDtypeStruct((1, AF), jnp.float32),
            jax.ShapeDtypeStruct((1, AF), jnp.float32),
        ],
    )(z)


def _final_body(a_ref, z_ref, s_ref, q_ref, gm_ref, bt_ref, o_ref):
    mean = s_ref[...] / N
    var = q_ref[...] / N - mean * mean
    scale = gm_ref[...] * lax.rsqrt(var + 1e-5)
    shift = bt_ref[...] - mean * scale
    o_ref[...] = _softplus(a_ref[...] + z_ref[...] * scale + shift)


def _finalize(a, z, s, q, gamma2, beta2):
    bn = 1000
    return pl.pallas_call(
        _final_body,
        grid=(N // bn,),
        in_specs=[
            pl.BlockSpec((bn, AF), lambda i: (i, 0)),
            pl.BlockSpec((bn, AF), lambda i: (i, 0)),
            pl.BlockSpec((1, AF), lambda i: (0, 0)),
            pl.BlockSpec((1, AF), lambda i: (0, 0)),
            pl.BlockSpec((1, AF), lambda i: (0, 0)),
            pl.BlockSpec((1, AF), lambda i: (0, 0)),
        ],
        out_specs=pl.BlockSpec((bn, AF), lambda i: (i, 0)),
        out_shape=jax.ShapeDtypeStruct((N, AF), jnp.float32),
    )(a, z, s, q, gamma2, beta2)


def kernel(atom_in_fea, nbr_fea, nbr_fea_idx, batch, W, b, gamma1, beta1,
           gamma2, beta2):
    del batch
    w1 = W[:AF, :]
    w2 = W[AF:2 * AF, :]
    w3 = W[2 * AF:, :]
    row = nbr_fea_idx[0]
    col = nbr_fea_idx[1]

    p1, p2 = _project(atom_in_fea, w1, w2)
    g1, g2 = _edge_gather(p1, p2, row, col)
    x, s1, q1 = _fuse_stats(g1, g2, nbr_fea, w3, b.reshape(1, H))
    y = _apply_act(x, s1, q1, gamma1.reshape(1, H), beta1.reshape(1, H))
    z = _segment_sum(y, col)[:N]
    s2, q2 = _z_stats(z)
    out = _finalize(atom_in_fea, z, s2, q2, gamma2.reshape(1, AF),
                    beta2.reshape(1, AF))
    return out


# speedup vs baseline: 3.2316x; 3.2316x over previous
import functools

import jax
import jax.numpy as jnp
from jax import lax
from jax.experimental import pallas as pl
from jax.experimental.pallas import tpu as pltpu
from jax.experimental.pallas import tpu_sc as plsc

N = 10000
E = 160000
AF = 256
NF = 16
H = 2 * AF

NC = 2
NS = 16
NW = NC * NS
EW = E // NW
CHUNK = 40
NCHUNK = EW // CHUNK
Z_PAD = 10240
ROWS_PER_TILE = Z_PAD // NS


def _proj_body(a_ref, w1_ref, w2_ref, o1_ref, o2_ref):
    a = a_ref[...]
    o1_ref[...] = jnp.dot(a, w1_ref[...], preferred_element_type=jnp.float32)
    o2_ref[...] = jnp.dot(a, w2_ref[...], preferred_element_type=jnp.float32)


def _project(a, w1, w2):
    bn = 1000
    return pl.pallas_call(
        _proj_body,
        grid=(N // bn,),
        in_specs=[
            pl.BlockSpec((bn, AF), lambda i: (i, 0)),
            pl.BlockSpec((AF, H), lambda i: (0, 0)),
            pl.BlockSpec((AF, H), lambda i: (0, 0)),
        ],
        out_specs=[
            pl.BlockSpec((bn, H), lambda i: (i, 0)),
            pl.BlockSpec((bn, H), lambda i: (i, 0)),
        ],
        out_shape=[
            jax.ShapeDtypeStruct((N, H), jnp.float32),
            jax.ShapeDtypeStruct((N, H), jnp.float32),
        ],
    )(a, w1, w2)


NBUF_G = 2
NRND_G = NCHUNK // NBUF_G


def _gather_body(p1_hbm, p2_hbm, row_hbm, col_hbm, g1_hbm, g2_hbm,
                 *bufs_and_sems):
    idx1 = bufs_and_sems[0:NBUF_G]
    idx2 = bufs_and_sems[NBUF_G:2 * NBUF_G]
    buf1 = bufs_and_sems[2 * NBUF_G:3 * NBUF_G]
    buf2 = bufs_and_sems[3 * NBUF_G:4 * NBUF_G]
    semi = bufs_and_sems[4 * NBUF_G:5 * NBUF_G]
    semg = bufs_and_sems[5 * NBUF_G:6 * NBUF_G]
    semw = bufs_and_sems[6 * NBUF_G:7 * NBUF_G]
    cid = lax.axis_index("c")
    sid = lax.axis_index("s")
    wid = sid * NC + cid
    base0 = wid * EW

    def issue_idx(i, b):
        base = base0 + i * CHUNK
        pltpu.async_copy(row_hbm.at[pl.ds(base, CHUNK)], idx1[b], semi[b])
        pltpu.async_copy(col_hbm.at[pl.ds(base, CHUNK)], idx2[b], semi[b])

    def wait_idx(i, b):
        base = base0 + i * CHUNK
        pltpu.make_async_copy(row_hbm.at[pl.ds(base, CHUNK)], idx1[b],
                              semi[b]).wait()
        pltpu.make_async_copy(col_hbm.at[pl.ds(base, CHUNK)], idx2[b],
                              semi[b]).wait()

    def wait_wb(i, b):
        base = base0 + i * CHUNK
        pltpu.make_async_copy(buf1[b], g1_hbm.at[pl.ds(base, CHUNK)],
                              semw[b]).wait()
        pltpu.make_async_copy(buf2[b], g2_hbm.at[pl.ds(base, CHUNK)],
                              semw[b]).wait()

    for b in range(NBUF_G):
        issue_idx(b, b)

    @pl.loop(0, NRND_G)
    def _(g):
        for b in range(NBUF_G):
            i = g * NBUF_G + b
            wait_idx(i, b)

            @pl.when(g > 0)
            def _():
                wait_wb(i - NBUF_G, b)
            pltpu.async_copy(p1_hbm.at[idx1[b]], buf1[b], semg[b])
            pltpu.async_copy(p2_hbm.at[idx2[b]], buf2[b], semg[b])

        for b in range(NBUF_G):
            i = g * NBUF_G + b
            pltpu.make_async_copy(p1_hbm.at[idx1[b]], buf1[b], semg[b]).wait()
            pltpu.make_async_copy(p2_hbm.at[idx2[b]], buf2[b], semg[b]).wait()
            base = base0 + i * CHUNK
            pltpu.async_copy(buf1[b], g1_hbm.at[pl.ds(base, CHUNK)], semw[b])
            pltpu.async_copy(buf2[b], g2_hbm.at[pl.ds(base, CHUNK)], semw[b])

            @pl.when(i + NBUF_G < NBUF_G * NRND_G)
            def _():
                issue_idx(i + NBUF_G, b)

    for b in range(NBUF_G):
        wait_wb(NBUF_G * NRND_G - NBUF_G + b, b)

    i = NBUF_G * NRND_G
    issue_idx(i, 0)
    wait_idx(i, 0)
    d1 = pltpu.async_copy(p1_hbm.at[idx1[0]], buf1[0], semg[0])
    d2 = pltpu.async_copy(p2_hbm.at[idx2[0]], buf2[0], semg[0])
    d1.wait()
    d2.wait()
    base = base0 + i * CHUNK
    pltpu.sync_copy(buf1[0], g1_hbm.at[pl.ds(base, CHUNK)])
    pltpu.sync_copy(buf2[0], g2_hbm.at[pl.ds(base, CHUNK)])


def _edge_gather(p1, p2, row, col):
    mesh = plsc.VectorSubcoreMesh(core_axis_name="c", subcore_axis_name="s")
    return pl.kernel(
        _gather_body,
        out_type=[
            jax.ShapeDtypeStruct((E, H), jnp.float32),
            jax.ShapeDtypeStruct((E, H), jnp.float32),
        ],
        mesh=mesh,
        scratch_types=(
            [pltpu.VMEM((CHUNK,), jnp.int32) for _ in range(2 * NBUF_G)] +
            [pltpu.VMEM((CHUNK, H), jnp.float32) for _ in range(2 * NBUF_G)] +
            [pltpu.SemaphoreType.DMA for _ in range(3 * NBUF_G)]
        ),
    )(p1, p2, row, col)


def _fuse_body(g1_ref, g2_ref, nbr_ref, w3_ref, b_ref, x_ref, s_ref, q_ref):
    x = g1_ref[...] + g2_ref[...]
    x = x + jnp.dot(nbr_ref[...], w3_ref[...], preferred_element_type=jnp.float32)
    x = x + b_ref[...]
    x_ref[...] = x

    @pl.when(pl.program_id(0) == 0)
    def _():
        s_ref[...] = jnp.zeros_like(s_ref)
        q_ref[...] = jnp.zeros_like(q_ref)

    s_ref[...] += jnp.sum(x, axis=0, keepdims=True)
    q_ref[...] += jnp.sum(x * x, axis=0, keepdims=True)


def _fuse_stats(g1, g2, nbr, w3, b):
    be = 1000
    return pl.pallas_call(
        _fuse_body,
        grid=(E // be,),
        in_specs=[
            pl.BlockSpec((be, H), lambda i: (i, 0)),
            pl.BlockSpec((be, H), lambda i: (i, 0)),
            pl.BlockSpec((be, NF), lambda i: (i, 0)),
            pl.BlockSpec((NF, H), lambda i: (0, 0)),
            pl.BlockSpec((1, H), lambda i: (0, 0)),
        ],
        out_specs=[
            pl.BlockSpec((be, H), lambda i: (i, 0)),
            pl.BlockSpec((1, H), lambda i: (0, 0)),
            pl.BlockSpec((1, H), lambda i: (0, 0)),
        ],
        out_shape=[
            jax.ShapeDtypeStruct((E, H), jnp.float32),
            jax.ShapeDtypeStruct((1, H), jnp.float32),
            jax.ShapeDtypeStruct((1, H), jnp.float32),
        ],
    )(g1, g2, nbr, w3, b)


def _softplus(x):
    return jnp.maximum(x, 0.0) + jnp.log1p(jnp.exp(-jnp.abs(x)))


def _apply_body(x_ref, s_ref, q_ref, gm_ref, bt_ref, y0_ref, y1_ref):
    mean = s_ref[...] / E
    var = q_ref[...] / E - mean * mean
    scale = gm_ref[...] * lax.rsqrt(var + 1e-5)
    shift = bt_ref[...] - mean * scale
    xb = x_ref[...] * scale + shift
    filt = jax.nn.sigmoid(xb[:, :AF])
    core = _softplus(xb[:, AF:])
    y = filt * core
    y0_ref[...] = y[:, :AF // 2]
    y1_ref[...] = y[:, AF // 2:]


def _apply_act(x, s, q, gamma1, beta1):
    be = 1000
    return pl.pallas_call(
        _apply_body,
        grid=(E // be,),
        in_specs=[
            pl.BlockSpec((be, H), lambda i: (i, 0)),
            pl.BlockSpec((1, H), lambda i: (0, 0)),
            pl.BlockSpec((1, H), lambda i: (0, 0)),
            pl.BlockSpec((1, H), lambda i: (0, 0)),
            pl.BlockSpec((1, H), lambda i: (0, 0)),
        ],
        out_specs=[
            pl.BlockSpec((be, AF // 2), lambda i: (i, 0)),
            pl.BlockSpec((be, AF // 2), lambda i: (i, 0)),
        ],
        out_shape=[
            jax.ShapeDtypeStruct((E, AF // 2), jnp.float32),
            jax.ShapeDtypeStruct((E, AF // 2), jnp.float32),
        ],
    )(x, s, q, gamma1, beta1)


CH_S = 64
NCH_S = E // CH_S
NBUF_S = 5


def _scatter_body(y0_hbm, y1_hbm, col_hbm, zeros_hbm, z_hbm, acc_sh,
                  *bufs_and_sems):
    idxb = bufs_and_sems[0:NBUF_S]
    ybuf = bufs_and_sems[NBUF_S:2 * NBUF_S]
    semi = bufs_and_sems[2 * NBUF_S:3 * NBUF_S]
    semy = bufs_and_sems[3 * NBUF_S:4 * NBUF_S]
    sems = bufs_and_sems[4 * NBUF_S:5 * NBUF_S]
    cid = lax.axis_index("c")
    sid = lax.axis_index("s")
    pltpu.sync_copy(zeros_hbm, acc_sh.at[pl.ds(sid * ROWS_PER_TILE, ROWS_PER_TILE)])
    plsc.subcore_barrier()

    def run(y_hbm):
        def issue_loads(i, b):
            pltpu.async_copy(col_hbm.at[pl.ds(i * CH_S, CH_S)], idxb[b], semi[b])
            pltpu.async_copy(y_hbm.at[pl.ds(i * CH_S, CH_S)], ybuf[b], semy[b])

        def wait_loads(i, b):
            pltpu.make_async_copy(
                col_hbm.at[pl.ds(i * CH_S, CH_S)], idxb[b], semi[b]).wait()
            pltpu.make_async_copy(
                y_hbm.at[pl.ds(i * CH_S, CH_S)], ybuf[b], semy[b]).wait()

        for b in range(NBUF_S):
            issue_loads(b, b)

        @pl.loop(0, NCH_S // NBUF_S)
        def _(g):
            for b in range(NBUF_S):
                i = g * NBUF_S + b

                @pl.when(g > 0)
                def _():
                    pltpu.make_async_copy(ybuf[b], acc_sh.at[idxb[b]],
                                          sems[b]).wait()
                    issue_loads(i, b)

            for b in range(NBUF_S):
                i = g * NBUF_S + b
                wait_loads(i, b)
                pltpu.async_copy(ybuf[b], acc_sh.at[idxb[b]], sems[b], add=True)

        for b in range(NBUF_S):
            pltpu.make_async_copy(ybuf[b], acc_sh.at[idxb[b]], sems[b]).wait()

    @pl.when((sid == 0) & (cid == 0))
    def _():
        run(y0_hbm)

    @pl.when((sid == 0) & (cid == 1))
    def _():
        run(y1_hbm)

    plsc.subcore_barrier()
    pltpu.sync_copy(
        acc_sh.at[pl.ds(sid * ROWS_PER_TILE, ROWS_PER_TILE)],
        z_hbm.at[pl.ds(sid * ROWS_PER_TILE, ROWS_PER_TILE),
                 pl.ds(cid * (AF // NC), AF // NC)],
    )


def _segment_sum(y0, y1, col):
    mesh = plsc.VectorSubcoreMesh(core_axis_name="c", subcore_axis_name="s")
    zeros = jnp.zeros((ROWS_PER_TILE, AF // NC), jnp.float32)
    return pl.kernel(
        _scatter_body,
        out_type=jax.ShapeDtypeStruct((Z_PAD, AF), jnp.float32),
        mesh=mesh,
        scratch_types=(
            [pltpu.VMEM_SHARED((Z_PAD, AF // NC), jnp.float32)] +
            [pltpu.VMEM((CH_S,), jnp.int32) for _ in range(NBUF_S)] +
            [pltpu.VMEM((CH_S, AF // NC), jnp.float32) for _ in range(NBUF_S)] +
            [pltpu.SemaphoreType.DMA for _ in range(3 * NBUF_S)]
        ),
    )(y0, y1, col, zeros)


def _zstat_body(z_ref, s_ref, q_ref):
    z = z_ref[...]

    @pl.when(pl.program_id(0) == 0)
    def _():
        s_ref[...] = jnp.zeros_like(s_ref)
        q_ref[...] = jnp.zeros_like(q_ref)

    s_ref[...] += jnp.sum(z, axis=0, keepdims=True)
    q_ref[...] += jnp.sum(z * z, axis=0, keepdims=True)


def _z_stats(z):
    bn = 1000
    return pl.pallas_call(
        _zstat_body,
        grid=(N // bn,),
        in_specs=[pl.BlockSpec((bn, AF), lambda i: (i, 0))],
        out_specs=[
            pl.BlockSpec((1, AF), lambda i: (0, 0)),
            pl.BlockSpec((1, AF), lambda i: (0, 0)),
        ],
        out_shape=[
            jax.ShapeDtypeStruct((1, AF), jnp.float32),
            jax.ShapeDtypeStruct((1, AF), jnp.float32),
        ],
    )(z)


def _final_body(a_ref, z_ref, s_ref, q_ref, gm_ref, bt_ref, o_ref):
    mean = s_ref[...] / N
    var = q_ref[...] / N - mean * mean
    scale = gm_ref[...] * lax.rsqrt(var + 1e-5)
    shift = bt_ref[...] - mean * scale
    o_ref[...] = _softplus(a_ref[...] + z_ref[...] * scale + shift)


def _finalize(a, z, s, q, gamma2, beta2):
    bn = 1000
    return pl.pallas_call(
        _final_body,
        grid=(N // bn,),
        in_specs=[
            pl.BlockSpec((bn, AF), lambda i: (i, 0)),
            pl.BlockSpec((bn, AF), lambda i: (i, 0)),
            pl.BlockSpec((1, AF), lambda i: (0, 0)),
            pl.BlockSpec((1, AF), lambda i: (0, 0)),
            pl.BlockSpec((1, AF), lambda i: (0, 0)),
            pl.BlockSpec((1, AF), lambda i: (0, 0)),
        ],
        out_specs=pl.BlockSpec((bn, AF), lambda i: (i, 0)),
        out_shape=jax.ShapeDtypeStruct((N, AF), jnp.float32),
    )(a, z, s, q, gamma2, beta2)


def kernel(atom_in_fea, nbr_fea, nbr_fea_idx, batch, W, b, gamma1, beta1,
           gamma2, beta2):
    del batch
    w1 = W[:AF, :]
    w2 = W[AF:2 * AF, :]
    w3 = W[2 * AF:, :]
    row = nbr_fea_idx[0]
    col = nbr_fea_idx[1]

    p1, p2 = _project(atom_in_fea, w1, w2)
    g1, g2 = _edge_gather(p1, p2, row, col)
    x, s1, q1 = _fuse_stats(g1, g2, nbr_fea, w3, b.reshape(1, H))
    y0, y1 = _apply_act(x, s1, q1, gamma1.reshape(1, H), beta1.reshape(1, H))
    z = _segment_sum(y0, y1, col)[:N]
    s2, q2 = _z_stats(z)
    out = _finalize(atom_in_fea, z, s2, q2, gamma2.reshape(1, AF),
                    beta2.reshape(1, AF))
    return out
